# Initial kernel scaffold; baseline (speedup 1.0000x reference)
import jax
import jax.numpy as jnp
from jax.experimental import pallas as pl


def kernel(X, edge_index, W1, b1, W2, b2, a):
    raise NotImplementedError("write your pallas kernel here")


# speedup vs baseline: 1.0046x; 1.0046x over previous
import functools

import jax
import jax.numpy as jnp
from jax.experimental import pallas as pl

N = 10000
HID = 128
ROW_BLK = 400


def _leaky(x):
    return jnp.where(x >= 0, x, 0.01 * x)


def _tail_kernel(x_ref, chs_ref, w1_ref, b1_ref, w2_ref, b2_ref, a_ref, out_ref):
    x = x_ref[...]
    ah = a_ref[0:HID, 0]
    al = a_ref[HID:2 * HID, 0]
    u = jnp.dot(x, ah[:, None], preferred_element_type=jnp.float32)
    es = []
    for i in range(6):
        ci = chs_ref[i]
        vi = jnp.dot(ci, al[:, None], preferred_element_type=jnp.float32)
        es.append(_leaky(u + vi))
    e = jnp.concatenate(es, axis=1)
    m = jnp.max(e, axis=1, keepdims=True)
    w = jnp.exp(e - m)
    att = w / jnp.sum(w, axis=1, keepdims=True)
    hp = jnp.zeros_like(x)
    for i in range(6):
        hp = hp + att[:, i:i + 1] * chs_ref[i]
    t = jax.lax.dot_general(hp, w1_ref[...], (((1,), (1,)), ((), ())),
                            preferred_element_type=jnp.float32) + b1_ref[...][None, :]
    t = jnp.maximum(t, 0.0)
    out = jax.lax.dot_general(t, w2_ref[...], (((1,), (1,)), ((), ())),
                              preferred_element_type=jnp.float32) + b2_ref[...][None, :]
    out_ref[...] = out


def _tail(X, chs, W1, b1, W2, b2, a):
    grid = (N // ROW_BLK,)
    return pl.pallas_call(
        _tail_kernel,
        grid=grid,
        in_specs=[
            pl.BlockSpec((ROW_BLK, HID), lambda i: (i, 0)),
            pl.BlockSpec((6, ROW_BLK, HID), lambda i: (0, i, 0)),
            pl.BlockSpec((HID, HID), lambda i: (0, 0)),
            pl.BlockSpec((HID,), lambda i: (0,)),
            pl.BlockSpec((HID, HID), lambda i: (0, 0)),
            pl.BlockSpec((HID,), lambda i: (0,)),
            pl.BlockSpec((2 * HID, 1), lambda i: (0, 0)),
        ],
        out_specs=pl.BlockSpec((ROW_BLK, HID), lambda i: (i, 0)),
        out_shape=jax.ShapeDtypeStruct((N, HID), jnp.float32),
    )(X, chs, W1, b1, W2, b2, a)


def kernel(X, edge_index, W1, b1, W2, b2, a):
    row = edge_index[0]
    col = edge_index[1]

    def spmm_W(x):
        return jax.ops.segment_sum(x[col], row, num_segments=N)

    deg_gcn = jnp.zeros((N,), dtype=jnp.float32).at[col].add(1.0) + 1.0
    Dg = jnp.power(deg_gcn, -0.5)[:, None]
    feat = X
    gcn_list = []
    for _ in range(3):
        feat = feat * Dg
        feat = spmm_W(feat) + feat
        feat = feat * Dg
        gcn_list.append(feat)
    h_A = _leaky(gcn_list[0])
    h_A2 = _leaky(gcn_list[1])
    h_A3 = _leaky(gcn_list[2])

    deg_w = jnp.zeros((N,), dtype=jnp.float32).at[col].add(1.0)
    Dw = jnp.power(deg_w, -1.0)[:, None]
    fp = X
    sct = []
    for _ in range(4):
        fp = 0.5 * fp + 0.5 * spmm_W(Dw * fp)
        sct.append(fp)
    s1 = jnp.abs(sct[0] - sct[1])
    s2 = jnp.abs(sct[1] - sct[2])
    s3 = jnp.abs(sct[2] - sct[3])

    chs = jnp.stack([h_A, h_A2, h_A3, s1, s2, s3], axis=0)
    return _tail(X, chs, W1, b1, W2, b2, a)


# speedup vs baseline: 1.9496x; 1.9407x over previous
import functools

import jax
import jax.numpy as jnp
from jax import lax
from jax.experimental import pallas as pl
from jax.experimental.pallas import tpu as pltpu
from jax.experimental.pallas import tpu_sc as plsc

N = 10000
HID = 128
E = 320000
NTILES = 32
EPT = E // NTILES
CHUNK = 128
NJ = 80
EPAD = NJ * CHUNK
NPAD = 10240
SLAB = NPAD // 16
NDEG = 10016
ROW_BLK = 1000


def _spmm_body(y_hbm, col_hbm, row_hbm, zeros_hbm, out_hbm,
               acc, col_v, row_v, gbuf, sem):
    c = lax.axis_index("c")
    s = lax.axis_index("s")
    w = c * 16 + s
    pltpu.sync_copy(col_hbm.at[w], col_v)
    pltpu.sync_copy(row_hbm.at[w], row_v)
    pltpu.sync_copy(zeros_hbm, acc.at[pl.ds(s * SLAB, SLAB)])
    plsc.subcore_barrier()

    def body(j, carry):
        pltpu.async_copy(y_hbm.at[col_v.at[j]], gbuf, sem).wait()
        pltpu.sync_copy(gbuf, acc.at[row_v.at[j]], add=True)
        return carry

    lax.fori_loop(0, NJ, body, 0)
    plsc.subcore_barrier()
    pltpu.sync_copy(acc.at[pl.ds(s * SLAB, SLAB)],
                    out_hbm.at[c, pl.ds(s * SLAB, SLAB)])


@functools.cache
def _get_spmm():
    return pl.kernel(
        _spmm_body,
        out_type=jax.ShapeDtypeStruct((2, NPAD, HID), jnp.float32),
        mesh=plsc.VectorSubcoreMesh(core_axis_name="c", subcore_axis_name="s",
                                    num_cores=2, num_subcores=16),
        scratch_types=[
            pltpu.VMEM_SHARED((NPAD, HID), jnp.float32),
            pltpu.VMEM((NJ, CHUNK), jnp.int32),
            pltpu.VMEM((NJ, CHUNK), jnp.int32),
            pltpu.VMEM((CHUNK, HID), jnp.float32),
            pltpu.SemaphoreType.DMA,
        ],
    )


DEGW = 16


def _deg_body(col_hbm, ones_hbm, zeros_hbm, out_hbm, dacc, col_v, ones_v):
    c = lax.axis_index("c")
    s = lax.axis_index("s")
    w = c * 16 + s
    pltpu.sync_copy(col_hbm.at[w], col_v)
    pltpu.sync_copy(ones_hbm, ones_v)
    pltpu.sync_copy(zeros_hbm, dacc.at[pl.ds(s * SLAB, SLAB)])
    plsc.subcore_barrier()

    def body(j, carry):
        pltpu.sync_copy(ones_v, dacc.at[col_v.at[j]], add=True)
        return carry

    lax.fori_loop(0, NJ, body, 0)
    plsc.subcore_barrier()
    pltpu.sync_copy(dacc.at[pl.ds(s * SLAB, SLAB)],
                    out_hbm.at[c, pl.ds(s * SLAB, SLAB)])


@functools.cache
def _get_deg():
    return pl.kernel(
        _deg_body,
        out_type=jax.ShapeDtypeStruct((2, NPAD, DEGW), jnp.float32),
        mesh=plsc.VectorSubcoreMesh(core_axis_name="c", subcore_axis_name="s",
                                    num_cores=2, num_subcores=16),
        scratch_types=[
            pltpu.VMEM_SHARED((NPAD, DEGW), jnp.float32),
            pltpu.VMEM((NJ, CHUNK), jnp.int32),
            pltpu.VMEM((CHUNK, DEGW), jnp.float32),
        ],
    )


def _leaky(x):
    return jnp.where(x >= 0, x, 0.01 * x)


def _degsum_body(degp_ref, out_ref):
    out_ref[...] = jnp.sum(degp_ref[...], axis=(0, 2))[:, None] * (1.0 / DEGW)


def _degsum(degp):
    return pl.pallas_call(
        _degsum_body,
        grid=(1,),
        in_specs=[pl.BlockSpec((2, NPAD, DEGW), lambda i: (0, 0, 0))],
        out_specs=pl.BlockSpec((NPAD, 1), lambda i: (0, 0)),
        out_shape=jax.ShapeDtypeStruct((NPAD, 1), jnp.float32),
    )(degp)


def _prep_body(x_ref, deg_ref, dg_ref, dw_ref, y0_ref, z0_ref):
    deg = deg_ref[...]
    dg = lax.rsqrt(deg + 1.0)
    dw = 1.0 / deg
    dg_ref[...] = dg
    dw_ref[...] = dw
    x = x_ref[...]
    y0_ref[...] = dg * x
    z0_ref[...] = dw * x


def _prep(X, deg):
    grid = (N // ROW_BLK,)
    return pl.pallas_call(
        _prep_body,
        grid=grid,
        in_specs=[
            pl.BlockSpec((ROW_BLK, HID), lambda i: (i, 0)),
            pl.BlockSpec((ROW_BLK, 1), lambda i: (i, 0)),
        ],
        out_specs=[
            pl.BlockSpec((ROW_BLK, 1), lambda i: (i, 0)),
            pl.BlockSpec((ROW_BLK, 1), lambda i: (i, 0)),
            pl.BlockSpec((ROW_BLK, HID), lambda i: (i, 0)),
            pl.BlockSpec((ROW_BLK, HID), lambda i: (i, 0)),
        ],
        out_shape=[
            jax.ShapeDtypeStruct((N, 1), jnp.float32),
            jax.ShapeDtypeStruct((N, 1), jnp.float32),
            jax.ShapeDtypeStruct((N, HID), jnp.float32),
            jax.ShapeDtypeStruct((N, HID), jnp.float32),
        ],
    )(X, deg)


def _gcn_body(p_ref, y_ref, dg_ref, ynext_ref, h_ref):
    dg = dg_ref[...]
    feat = dg * (p_ref[0] + p_ref[1] + y_ref[...])
    h_ref[...] = _leaky(feat)
    ynext_ref[...] = dg * feat


def _gcn_update(P, y, Dg):
    grid = (N // ROW_BLK,)
    return pl.pallas_call(
        _gcn_body,
        grid=grid,
        in_specs=[
            pl.BlockSpec((2, ROW_BLK, HID), lambda i: (0, i, 0)),
            pl.BlockSpec((ROW_BLK, HID), lambda i: (i, 0)),
            pl.BlockSpec((ROW_BLK, 1), lambda i: (i, 0)),
        ],
        out_specs=[
            pl.BlockSpec((ROW_BLK, HID), lambda i: (i, 0)),
            pl.BlockSpec((ROW_BLK, HID), lambda i: (i, 0)),
        ],
        out_shape=[
            jax.ShapeDtypeStruct((N, HID), jnp.float32),
            jax.ShapeDtypeStruct((N, HID), jnp.float32),
        ],
    )(P, y, Dg)


def _sct_body(p_ref, fp_ref, dw_ref, fpn_ref, zn_ref):
    fpn = 0.5 * fp_ref[...] + 0.5 * (p_ref[0] + p_ref[1])
    fpn_ref[...] = fpn
    zn_ref[...] = dw_ref[...] * fpn


def _sct_update(P, fp, Dw):
    grid = (N // ROW_BLK,)
    return pl.pallas_call(
        _sct_body,
        grid=grid,
        in_specs=[
            pl.BlockSpec((2, ROW_BLK, HID), lambda i: (0, i, 0)),
            pl.BlockSpec((ROW_BLK, HID), lambda i: (i, 0)),
            pl.BlockSpec((ROW_BLK, 1), lambda i: (i, 0)),
        ],
        out_specs=[
            pl.BlockSpec((ROW_BLK, HID), lambda i: (i, 0)),
            pl.BlockSpec((ROW_BLK, HID), lambda i: (i, 0)),
        ],
        out_shape=[
            jax.ShapeDtypeStruct((N, HID), jnp.float32),
            jax.ShapeDtypeStruct((N, HID), jnp.float32),
        ],
    )(P, fp, Dw)


def _tail_body(x_ref, h1_ref, h2_ref, h3_ref, f1_ref, f2_ref, f3_ref, f4_ref,
               w1_ref, b1_ref, w2_ref, b2_ref, a_ref, out_ref):
    x = x_ref[...]
    chs = [
        h1_ref[...], h2_ref[...], h3_ref[...],
        jnp.abs(f1_ref[...] - f2_ref[...]),
        jnp.abs(f2_ref[...] - f3_ref[...]),
        jnp.abs(f3_ref[...] - f4_ref[...]),
    ]
    ah = a_ref[0:HID, 0]
    al = a_ref[HID:2 * HID, 0]
    u = jnp.dot(x, ah[:, None], preferred_element_type=jnp.float32)
    es = []
    for ci in chs:
        vi = jnp.dot(ci, al[:, None], preferred_element_type=jnp.float32)
        es.append(_leaky(u + vi))
    e = jnp.concatenate(es, axis=1)
    m = jnp.max(e, axis=1, keepdims=True)
    w = jnp.exp(e - m)
    att = w / jnp.sum(w, axis=1, keepdims=True)
    hp = jnp.zeros_like(x)
    for i, ci in enumerate(chs):
        hp = hp + att[:, i:i + 1] * ci
    t = lax.dot_general(hp, w1_ref[...], (((1,), (1,)), ((), ())),
                        preferred_element_type=jnp.float32) + b1_ref[...][None, :]
    t = jnp.maximum(t, 0.0)
    out_ref[...] = lax.dot_general(t, w2_ref[...], (((1,), (1,)), ((), ())),
                                   preferred_element_type=jnp.float32) + b2_ref[...][None, :]


def _tail(X, h1, h2, h3, f1, f2, f3, f4, W1, b1, W2, b2, a):
    grid = (N // ROW_BLK,)
    rb = pl.BlockSpec((ROW_BLK, HID), lambda i: (i, 0))
    return pl.pallas_call(
        _tail_body,
        grid=grid,
        in_specs=[rb] * 8 + [
            pl.BlockSpec((HID, HID), lambda i: (0, 0)),
            pl.BlockSpec((HID,), lambda i: (0,)),
            pl.BlockSpec((HID, HID), lambda i: (0, 0)),
            pl.BlockSpec((HID,), lambda i: (0,)),
            pl.BlockSpec((2 * HID, 1), lambda i: (0, 0)),
        ],
        out_specs=rb,
        out_shape=jax.ShapeDtypeStruct((N, HID), jnp.float32),
    )(X, h1, h2, h3, f1, f2, f3, f4, W1, b1, W2, b2, a)


def kernel(X, edge_index, W1, b1, W2, b2, a):
    row = edge_index[0]
    col = edge_index[1]
    pad = EPAD - EPT
    col_r = col.reshape(NTILES, EPT)
    row_r = row.reshape(NTILES, EPT)
    col_g = jnp.pad(col_r, ((0, 0), (0, pad))).reshape(NTILES, NJ, CHUNK)
    row_g = jnp.pad(row_r, ((0, 0), (0, pad)),
                    constant_values=N).reshape(NTILES, NJ, CHUNK)
    col_d = jnp.pad(col_r, ((0, 0), (0, pad)),
                    constant_values=N).reshape(NTILES, NJ, CHUNK)
    zeros_slab = jnp.zeros((SLAB, HID), jnp.float32)
    ones_deg = jnp.ones((CHUNK, DEGW), jnp.float32)
    zeros_deg = jnp.zeros((SLAB, DEGW), jnp.float32)

    degp = _get_deg()(col_d, ones_deg, zeros_deg)
    deg = _degsum(degp)[:N]
    Dg, Dw, y, z = _prep(X, deg)

    spmm = _get_spmm()
    hs = []
    for _ in range(3):
        P = spmm(y, col_g, row_g, zeros_slab)
        y, h = _gcn_update(P[:, :N], y, Dg)
        hs.append(h)

    fp = X
    fs = []
    for _ in range(4):
        P = spmm(z, col_g, row_g, zeros_slab)
        fp, z = _sct_update(P[:, :N], fp, Dw)
        fs.append(fp)

    return _tail(X, hs[0], hs[1], hs[2], fs[0], fs[1], fs[2], fs[3],
                 W1, b1, W2, b2, a)


# speedup vs baseline: 2.0774x; 1.0655x over previous
import functools

import jax
import jax.numpy as jnp
from jax import lax
from jax.experimental import pallas as pl
from jax.experimental.pallas import tpu as pltpu
from jax.experimental.pallas import tpu_sc as plsc

N = 10000
HID = 128
E = 320000
NTILES = 32
EPT = E // NTILES
CHUNK = 128
NJ = 80
EPAD = NJ * CHUNK
NPAD = 10240
SLAB = NPAD // 16
NDEG = 10016
ROW_BLK = 1000


NBUF = 2
NGRP = NJ // NBUF


def _spmm_body(y_hbm, pk_hbm, zeros_hbm, out_hbm,
               acc, pk_v, cb0, rb0, cb1, rb1, gb0, gb1, gsems, ssems):
    cbufs = [cb0, cb1]
    rbufs = [rb0, rb1]
    gbufs = [gb0, gb1]
    c = lax.axis_index("c")
    s = lax.axis_index("s")
    w = c * 16 + s
    pltpu.sync_copy(pk_hbm.at[w], pk_v)
    pltpu.sync_copy(zeros_hbm, acc.at[pl.ds(s * SLAB, SLAB)])
    plsc.subcore_barrier()

    def decode(j, b):
        for k in range(CHUNK // 16):
            p = pk_v[j, pl.ds(k * 16, 16)]
            cbufs[b][pl.ds(k * 16, 16)] = p & 0x3FFF
            rbufs[b][pl.ds(k * 16, 16)] = lax.shift_right_logical(p, 14)

    for b in range(NBUF):
        decode(b, b)
        pltpu.async_copy(y_hbm.at[cbufs[b]], gbufs[b], gsems.at[b])

    def body(i, carry):
        for b in range(NBUF):
            pltpu.make_async_copy(y_hbm.at[cbufs[b]], gbufs[b],
                                  gsems.at[b]).wait()
            pltpu.async_copy(gbufs[b], acc.at[rbufs[b]], ssems.at[b],
                             add=True)
        for b in range(NBUF):
            pltpu.make_async_copy(gbufs[b], acc.at[rbufs[b]],
                                  ssems.at[b]).wait()

            @pl.when(i + 1 < NGRP)
            def _(i=i, b=b):
                decode(i * NBUF + b + NBUF, b)
                pltpu.async_copy(y_hbm.at[cbufs[b]], gbufs[b], gsems.at[b])
        return carry

    lax.fori_loop(0, NGRP, body, 0)
    plsc.subcore_barrier()
    pltpu.sync_copy(acc.at[pl.ds(s * SLAB, SLAB)],
                    out_hbm.at[c, pl.ds(s * SLAB, SLAB)])


@functools.cache
def _get_spmm():
    return pl.kernel(
        _spmm_body,
        out_type=jax.ShapeDtypeStruct((2, NPAD, HID), jnp.float32),
        mesh=plsc.VectorSubcoreMesh(core_axis_name="c", subcore_axis_name="s",
                                    num_cores=2, num_subcores=16),
        scratch_types=[
            pltpu.VMEM_SHARED((NPAD, HID), jnp.float32),
            pltpu.VMEM((NJ, CHUNK), jnp.int32),
            pltpu.VMEM((CHUNK,), jnp.int32),
            pltpu.VMEM((CHUNK,), jnp.int32),
            pltpu.VMEM((CHUNK,), jnp.int32),
            pltpu.VMEM((CHUNK,), jnp.int32),
            pltpu.VMEM((CHUNK, HID), jnp.float32),
            pltpu.VMEM((CHUNK, HID), jnp.float32),
            pltpu.SemaphoreType.DMA((NBUF,)),
            pltpu.SemaphoreType.DMA((NBUF,)),
        ],
    )


DEGW = 16


def _deg_body(col_hbm, ones_hbm, zeros_hbm, out_hbm, dacc, col_v, ones_v):
    c = lax.axis_index("c")
    s = lax.axis_index("s")
    w = c * 16 + s
    pltpu.sync_copy(col_hbm.at[w], col_v)
    pltpu.sync_copy(ones_hbm, ones_v)
    pltpu.sync_copy(zeros_hbm, dacc.at[pl.ds(s * SLAB, SLAB)])
    plsc.subcore_barrier()

    def body(j, carry):
        pltpu.sync_copy(ones_v, dacc.at[col_v.at[j]], add=True)
        return carry

    lax.fori_loop(0, NJ, body, 0)
    plsc.subcore_barrier()
    pltpu.sync_copy(dacc.at[pl.ds(s * SLAB, SLAB)],
                    out_hbm.at[c, pl.ds(s * SLAB, SLAB)])


@functools.cache
def _get_deg():
    return pl.kernel(
        _deg_body,
        out_type=jax.ShapeDtypeStruct((2, NPAD, DEGW), jnp.float32),
        mesh=plsc.VectorSubcoreMesh(core_axis_name="c", subcore_axis_name="s",
                                    num_cores=2, num_subcores=16),
        scratch_types=[
            pltpu.VMEM_SHARED((NPAD, DEGW), jnp.float32),
            pltpu.VMEM((NJ, CHUNK), jnp.int32),
            pltpu.VMEM((CHUNK, DEGW), jnp.float32),
        ],
    )


def _leaky(x):
    return jnp.where(x >= 0, x, 0.01 * x)


def _degsum_body(degp_ref, out_ref):
    out_ref[...] = jnp.sum(degp_ref[...], axis=(0, 2))[:, None] * (1.0 / DEGW)


def _degsum(degp):
    return pl.pallas_call(
        _degsum_body,
        grid=(1,),
        in_specs=[pl.BlockSpec((2, NPAD, DEGW), lambda i: (0, 0, 0))],
        out_specs=pl.BlockSpec((NPAD, 1), lambda i: (0, 0)),
        out_shape=jax.ShapeDtypeStruct((NPAD, 1), jnp.float32),
    )(degp)


def _prep_body(x_ref, deg_ref, dg_ref, dw_ref, y0_ref, z0_ref):
    deg = deg_ref[...]
    dg = lax.rsqrt(deg + 1.0)
    dw = 1.0 / deg
    dg_ref[...] = dg
    dw_ref[...] = dw
    x = x_ref[...]
    y0_ref[...] = dg * x
    z0_ref[...] = dw * x


def _prep(X, deg):
    grid = (N // ROW_BLK,)
    return pl.pallas_call(
        _prep_body,
        grid=grid,
        in_specs=[
            pl.BlockSpec((ROW_BLK, HID), lambda i: (i, 0)),
            pl.BlockSpec((ROW_BLK, 1), lambda i: (i, 0)),
        ],
        out_specs=[
            pl.BlockSpec((ROW_BLK, 1), lambda i: (i, 0)),
            pl.BlockSpec((ROW_BLK, 1), lambda i: (i, 0)),
            pl.BlockSpec((ROW_BLK, HID), lambda i: (i, 0)),
            pl.BlockSpec((ROW_BLK, HID), lambda i: (i, 0)),
        ],
        out_shape=[
            jax.ShapeDtypeStruct((N, 1), jnp.float32),
            jax.ShapeDtypeStruct((N, 1), jnp.float32),
            jax.ShapeDtypeStruct((N, HID), jnp.float32),
            jax.ShapeDtypeStruct((N, HID), jnp.float32),
        ],
    )(X, deg)


def _gcn_body(p_ref, y_ref, dg_ref, ynext_ref, h_ref):
    dg = dg_ref[...]
    feat = dg * (p_ref[0] + p_ref[1] + y_ref[...])
    h_ref[...] = _leaky(feat)
    ynext_ref[...] = dg * feat


def _gcn_update(P, y, Dg):
    grid = (N // ROW_BLK,)
    return pl.pallas_call(
        _gcn_body,
        grid=grid,
        in_specs=[
            pl.BlockSpec((2, ROW_BLK, HID), lambda i: (0, i, 0)),
            pl.BlockSpec((ROW_BLK, HID), lambda i: (i, 0)),
            pl.BlockSpec((ROW_BLK, 1), lambda i: (i, 0)),
        ],
        out_specs=[
            pl.BlockSpec((ROW_BLK, HID), lambda i: (i, 0)),
            pl.BlockSpec((ROW_BLK, HID), lambda i: (i, 0)),
        ],
        out_shape=[
            jax.ShapeDtypeStruct((N, HID), jnp.float32),
            jax.ShapeDtypeStruct((N, HID), jnp.float32),
        ],
    )(P, y, Dg)


def _sct_body(p_ref, fp_ref, dw_ref, fpn_ref, zn_ref):
    fpn = 0.5 * fp_ref[...] + 0.5 * (p_ref[0] + p_ref[1])
    fpn_ref[...] = fpn
    zn_ref[...] = dw_ref[...] * fpn


def _sct_update(P, fp, Dw):
    grid = (N // ROW_BLK,)
    return pl.pallas_call(
        _sct_body,
        grid=grid,
        in_specs=[
            pl.BlockSpec((2, ROW_BLK, HID), lambda i: (0, i, 0)),
            pl.BlockSpec((ROW_BLK, HID), lambda i: (i, 0)),
            pl.BlockSpec((ROW_BLK, 1), lambda i: (i, 0)),
        ],
        out_specs=[
            pl.BlockSpec((ROW_BLK, HID), lambda i: (i, 0)),
            pl.BlockSpec((ROW_BLK, HID), lambda i: (i, 0)),
        ],
        out_shape=[
            jax.ShapeDtypeStruct((N, HID), jnp.float32),
            jax.ShapeDtypeStruct((N, HID), jnp.float32),
        ],
    )(P, fp, Dw)


def _tail_body(x_ref, h1_ref, h2_ref, h3_ref, f1_ref, f2_ref, f3_ref, f4_ref,
               w1_ref, b1_ref, w2_ref, b2_ref, a_ref, out_ref):
    x = x_ref[...]
    chs = [
        h1_ref[...], h2_ref[...], h3_ref[...],
        jnp.abs(f1_ref[...] - f2_ref[...]),
        jnp.abs(f2_ref[...] - f3_ref[...]),
        jnp.abs(f3_ref[...] - f4_ref[...]),
    ]
    ah = a_ref[0:HID, 0]
    al = a_ref[HID:2 * HID, 0]
    u = jnp.dot(x, ah[:, None], preferred_element_type=jnp.float32)
    es = []
    for ci in chs:
        vi = jnp.dot(ci, al[:, None], preferred_element_type=jnp.float32)
        es.append(_leaky(u + vi))
    e = jnp.concatenate(es, axis=1)
    m = jnp.max(e, axis=1, keepdims=True)
    w = jnp.exp(e - m)
    att = w / jnp.sum(w, axis=1, keepdims=True)
    hp = jnp.zeros_like(x)
    for i, ci in enumerate(chs):
        hp = hp + att[:, i:i + 1] * ci
    t = lax.dot_general(hp, w1_ref[...], (((1,), (1,)), ((), ())),
                        preferred_element_type=jnp.float32) + b1_ref[...][None, :]
    t = jnp.maximum(t, 0.0)
    out_ref[...] = lax.dot_general(t, w2_ref[...], (((1,), (1,)), ((), ())),
                                   preferred_element_type=jnp.float32) + b2_ref[...][None, :]


def _tail(X, h1, h2, h3, f1, f2, f3, f4, W1, b1, W2, b2, a):
    grid = (N // ROW_BLK,)
    rb = pl.BlockSpec((ROW_BLK, HID), lambda i: (i, 0))
    return pl.pallas_call(
        _tail_body,
        grid=grid,
        in_specs=[rb] * 8 + [
            pl.BlockSpec((HID, HID), lambda i: (0, 0)),
            pl.BlockSpec((HID,), lambda i: (0,)),
            pl.BlockSpec((HID, HID), lambda i: (0, 0)),
            pl.BlockSpec((HID,), lambda i: (0,)),
            pl.BlockSpec((2 * HID, 1), lambda i: (0, 0)),
        ],
        out_specs=rb,
        out_shape=jax.ShapeDtypeStruct((N, HID), jnp.float32),
    )(X, h1, h2, h3, f1, f2, f3, f4, W1, b1, W2, b2, a)


def kernel(X, edge_index, W1, b1, W2, b2, a):
    row = edge_index[0]
    col = edge_index[1]
    pad = EPAD - EPT
    col_r = col.reshape(NTILES, EPT)
    row_r = row.reshape(NTILES, EPT)
    col_g = jnp.pad(col_r, ((0, 0), (0, pad))).reshape(NTILES, NJ, CHUNK)
    row_g = jnp.pad(row_r, ((0, 0), (0, pad)),
                    constant_values=N).reshape(NTILES, NJ, CHUNK)
    pk_g = (row_g << 14) | col_g
    col_d = jnp.pad(col_r, ((0, 0), (0, pad)),
                    constant_values=N).reshape(NTILES, NJ, CHUNK)
    zeros_slab = jnp.zeros((SLAB, HID), jnp.float32)
    ones_deg = jnp.ones((CHUNK, DEGW), jnp.float32)
    zeros_deg = jnp.zeros((SLAB, DEGW), jnp.float32)

    degp = _get_deg()(col_d, ones_deg, zeros_deg)
    deg = _degsum(degp)[:N]
    Dg, Dw, y, z = _prep(X, deg)

    spmm = _get_spmm()
    hs = []
    for _ in range(3):
        P = spmm(y, pk_g, zeros_slab)
        y, h = _gcn_update(P[:, :N], y, Dg)
        hs.append(h)

    fp = X
    fs = []
    for _ in range(4):
        P = spmm(z, pk_g, zeros_slab)
        fp, z = _sct_update(P[:, :N], fp, Dw)
        fs.append(fp)

    return _tail(X, hs[0], hs[1], hs[2], fs[0], fs[1], fs[2], fs[3],
                 W1, b1, W2, b2, a)


# speedup vs baseline: 4.1340x; 1.9900x over previous
import functools

import jax
import jax.numpy as jnp
from jax import lax
from jax.experimental import pallas as pl
from jax.experimental.pallas import tpu as pltpu
from jax.experimental.pallas import tpu_sc as plsc

N = 10000
HID = 128
E = 320000
NTILES = 32
EPT = E // NTILES
CHUNK = 128
NJ = 80
HIDH = HID // 2
ESUB = E // 16
SNJ = 160
SCH = 64
SNC = 320
EPAD = NJ * CHUNK
NPAD = 10240
SLAB = NPAD // 16
NDEG = 10016
ROW_BLK = 1000


NBUF = 2
NGRP = SNC // NBUF


def _spmm_body(y_hbm, pk_hbm, zeros_hbm, out_hbm,
               ysp, acc, pk_v, cb0, rb0, cb1, rb1, gb0, gb1, gsems, ssems):
    cbufs = [cb0, cb1]
    rbufs = [rb0, rb1]
    gbufs = [gb0, gb1]
    c = lax.axis_index("c")
    s = lax.axis_index("s")
    pltpu.sync_copy(y_hbm.at[c, pl.ds(s * SLAB, SLAB)],
                    ysp.at[pl.ds(s * SLAB, SLAB)])
    pltpu.sync_copy(zeros_hbm, acc.at[pl.ds(s * SLAB, SLAB)])
    pltpu.sync_copy(pk_hbm.at[s], pk_v)
    plsc.subcore_barrier()

    def decode(j, b):
        base = (j % 2) * SCH
        for k in range(SCH // 16):
            p = pk_v[j // 2, pl.ds(base + k * 16, 16)]
            cbufs[b][pl.ds(k * 16, 16)] = p & 0x3FFF
            rbufs[b][pl.ds(k * 16, 16)] = lax.shift_right_logical(p, 14)

    for b in range(NBUF):
        decode(b, b)
        pltpu.async_copy(ysp.at[cbufs[b]], gbufs[b], gsems.at[b])

    def body(i, carry):
        for b in range(NBUF):
            pltpu.make_async_copy(ysp.at[cbufs[b]], gbufs[b],
                                  gsems.at[b]).wait()
            pltpu.async_copy(gbufs[b], acc.at[rbufs[b]], ssems.at[b],
                             add=True)
        for b in range(NBUF):
            pltpu.make_async_copy(gbufs[b], acc.at[rbufs[b]],
                                  ssems.at[b]).wait()

            @pl.when(i + 1 < NGRP)
            def _(i=i, b=b):
                decode(i * NBUF + b + NBUF, b)
                pltpu.async_copy(ysp.at[cbufs[b]], gbufs[b], gsems.at[b])
        return carry

    lax.fori_loop(0, NGRP, body, 0)
    plsc.subcore_barrier()
    pltpu.sync_copy(acc.at[pl.ds(s * SLAB, SLAB)],
                    out_hbm.at[c, pl.ds(s * SLAB, SLAB)])


@functools.cache
def _get_spmm():
    return pl.kernel(
        _spmm_body,
        out_type=jax.ShapeDtypeStruct((2, NPAD, HIDH), jnp.float32),
        mesh=plsc.VectorSubcoreMesh(core_axis_name="c", subcore_axis_name="s",
                                    num_cores=2, num_subcores=16),
        scratch_types=[
            pltpu.VMEM_SHARED((NPAD, HIDH), jnp.float32),
            pltpu.VMEM_SHARED((NPAD, HIDH), jnp.float32),
            pltpu.VMEM((SNJ, CHUNK), jnp.int32),
            pltpu.VMEM((SCH,), jnp.int32),
            pltpu.VMEM((SCH,), jnp.int32),
            pltpu.VMEM((SCH,), jnp.int32),
            pltpu.VMEM((SCH,), jnp.int32),
            pltpu.VMEM((SCH, HIDH), jnp.float32),
            pltpu.VMEM((SCH, HIDH), jnp.float32),
            pltpu.SemaphoreType.DMA((NBUF,)),
            pltpu.SemaphoreType.DMA((NBUF,)),
        ],
    )


DEGW = 16


def _deg_body(col_hbm, ones_hbm, zeros_hbm, out_hbm, dacc, col_v, ones_v):
    c = lax.axis_index("c")
    s = lax.axis_index("s")
    w = c * 16 + s
    pltpu.sync_copy(col_hbm.at[w], col_v)
    pltpu.sync_copy(ones_hbm, ones_v)
    pltpu.sync_copy(zeros_hbm, dacc.at[pl.ds(s * SLAB, SLAB)])
    plsc.subcore_barrier()

    def body(j, carry):
        pltpu.sync_copy(ones_v, dacc.at[col_v.at[j]], add=True)
        return carry

    lax.fori_loop(0, NJ, body, 0)
    plsc.subcore_barrier()
    pltpu.sync_copy(dacc.at[pl.ds(s * SLAB, SLAB)],
                    out_hbm.at[c, pl.ds(s * SLAB, SLAB)])


@functools.cache
def _get_deg():
    return pl.kernel(
        _deg_body,
        out_type=jax.ShapeDtypeStruct((2, NPAD, DEGW), jnp.float32),
        mesh=plsc.VectorSubcoreMesh(core_axis_name="c", subcore_axis_name="s",
                                    num_cores=2, num_subcores=16),
        scratch_types=[
            pltpu.VMEM_SHARED((NPAD, DEGW), jnp.float32),
            pltpu.VMEM((NJ, CHUNK), jnp.int32),
            pltpu.VMEM((CHUNK, DEGW), jnp.float32),
        ],
    )


def _leaky(x):
    return jnp.where(x >= 0, x, 0.01 * x)


def _degsum_body(degp_ref, out_ref):
    out_ref[...] = jnp.sum(degp_ref[...], axis=(0, 2))[:, None] * (1.0 / DEGW)


def _degsum(degp):
    return pl.pallas_call(
        _degsum_body,
        grid=(1,),
        in_specs=[pl.BlockSpec((2, NPAD, DEGW), lambda i: (0, 0, 0))],
        out_specs=pl.BlockSpec((NPAD, 1), lambda i: (0, 0)),
        out_shape=jax.ShapeDtypeStruct((NPAD, 1), jnp.float32),
    )(degp)


def _prep_body(x_ref, deg_ref, dg_ref, dw_ref, y0_ref, z0_ref):
    deg = deg_ref[...]
    dg = lax.rsqrt(deg + 1.0)
    dw = 1.0 / deg
    dg_ref[...] = dg
    dw_ref[...] = dw
    x = x_ref[...]
    y0 = dg * x
    z0 = dw * x
    y0_ref[0] = y0[:, :HIDH]
    y0_ref[1] = y0[:, HIDH:]
    z0_ref[0] = z0[:, :HIDH]
    z0_ref[1] = z0[:, HIDH:]


def _prep(X, deg):
    grid = (N // ROW_BLK,)
    return pl.pallas_call(
        _prep_body,
        grid=grid,
        in_specs=[
            pl.BlockSpec((ROW_BLK, HID), lambda i: (i, 0)),
            pl.BlockSpec((ROW_BLK, 1), lambda i: (i, 0)),
        ],
        out_specs=[
            pl.BlockSpec((ROW_BLK, 1), lambda i: (i, 0)),
            pl.BlockSpec((ROW_BLK, 1), lambda i: (i, 0)),
            pl.BlockSpec((2, ROW_BLK, HIDH), lambda i: (0, i, 0)),
            pl.BlockSpec((2, ROW_BLK, HIDH), lambda i: (0, i, 0)),
        ],
        out_shape=[
            jax.ShapeDtypeStruct((N, 1), jnp.float32),
            jax.ShapeDtypeStruct((N, 1), jnp.float32),
            jax.ShapeDtypeStruct((2, NPAD, HIDH), jnp.float32),
            jax.ShapeDtypeStruct((2, NPAD, HIDH), jnp.float32),
        ],
    )(X, deg)


def _gcn_body(p_ref, y_ref, dg_ref, ynext_ref, h_ref):
    dg = dg_ref[...]
    yfull = jnp.concatenate([y_ref[0], y_ref[1]], axis=1)
    feat = dg * (jnp.concatenate([p_ref[0], p_ref[1]], axis=1) + yfull)
    h_ref[...] = _leaky(feat)
    yn = dg * feat
    ynext_ref[0] = yn[:, :HIDH]
    ynext_ref[1] = yn[:, HIDH:]


def _gcn_update(P, y, Dg):
    grid = (N // ROW_BLK,)
    return pl.pallas_call(
        _gcn_body,
        grid=grid,
        in_specs=[
            pl.BlockSpec((2, ROW_BLK, HIDH), lambda i: (0, i, 0)),
            pl.BlockSpec((2, ROW_BLK, HIDH), lambda i: (0, i, 0)),
            pl.BlockSpec((ROW_BLK, 1), lambda i: (i, 0)),
        ],
        out_specs=[
            pl.BlockSpec((2, ROW_BLK, HIDH), lambda i: (0, i, 0)),
            pl.BlockSpec((ROW_BLK, HID), lambda i: (i, 0)),
        ],
        out_shape=[
            jax.ShapeDtypeStruct((2, NPAD, HIDH), jnp.float32),
            jax.ShapeDtypeStruct((N, HID), jnp.float32),
        ],
    )(P, y, Dg)


def _sct_body(p_ref, fp_ref, dw_ref, fpn_ref, zn_ref):
    fpn = 0.5 * fp_ref[...] + 0.5 * jnp.concatenate(
        [p_ref[0], p_ref[1]], axis=1)
    fpn_ref[...] = fpn
    zn = dw_ref[...] * fpn
    zn_ref[0] = zn[:, :HIDH]
    zn_ref[1] = zn[:, HIDH:]


def _sct_update(P, fp, Dw):
    grid = (N // ROW_BLK,)
    return pl.pallas_call(
        _sct_body,
        grid=grid,
        in_specs=[
            pl.BlockSpec((2, ROW_BLK, HIDH), lambda i: (0, i, 0)),
            pl.BlockSpec((ROW_BLK, HID), lambda i: (i, 0)),
            pl.BlockSpec((ROW_BLK, 1), lambda i: (i, 0)),
        ],
        out_specs=[
            pl.BlockSpec((ROW_BLK, HID), lambda i: (i, 0)),
            pl.BlockSpec((2, ROW_BLK, HIDH), lambda i: (0, i, 0)),
        ],
        out_shape=[
            jax.ShapeDtypeStruct((N, HID), jnp.float32),
            jax.ShapeDtypeStruct((2, NPAD, HIDH), jnp.float32),
        ],
    )(P, fp, Dw)


def _tail_body(x_ref, h1_ref, h2_ref, h3_ref, f1_ref, f2_ref, f3_ref, f4_ref,
               w1_ref, b1_ref, w2_ref, b2_ref, a_ref, out_ref):
    x = x_ref[...]
    chs = [
        h1_ref[...], h2_ref[...], h3_ref[...],
        jnp.abs(f1_ref[...] - f2_ref[...]),
        jnp.abs(f2_ref[...] - f3_ref[...]),
        jnp.abs(f3_ref[...] - f4_ref[...]),
    ]
    ah = a_ref[0:HID, 0]
    al = a_ref[HID:2 * HID, 0]
    u = jnp.dot(x, ah[:, None], preferred_element_type=jnp.float32)
    es = []
    for ci in chs:
        vi = jnp.dot(ci, al[:, None], preferred_element_type=jnp.float32)
        es.append(_leaky(u + vi))
    e = jnp.concatenate(es, axis=1)
    m = jnp.max(e, axis=1, keepdims=True)
    w = jnp.exp(e - m)
    att = w / jnp.sum(w, axis=1, keepdims=True)
    hp = jnp.zeros_like(x)
    for i, ci in enumerate(chs):
        hp = hp + att[:, i:i + 1] * ci
    t = lax.dot_general(hp, w1_ref[...], (((1,), (1,)), ((), ())),
                        preferred_element_type=jnp.float32) + b1_ref[...][None, :]
    t = jnp.maximum(t, 0.0)
    out_ref[...] = lax.dot_general(t, w2_ref[...], (((1,), (1,)), ((), ())),
                                   preferred_element_type=jnp.float32) + b2_ref[...][None, :]


def _tail(X, h1, h2, h3, f1, f2, f3, f4, W1, b1, W2, b2, a):
    grid = (N // ROW_BLK,)
    rb = pl.BlockSpec((ROW_BLK, HID), lambda i: (i, 0))
    return pl.pallas_call(
        _tail_body,
        grid=grid,
        in_specs=[rb] * 8 + [
            pl.BlockSpec((HID, HID), lambda i: (0, 0)),
            pl.BlockSpec((HID,), lambda i: (0,)),
            pl.BlockSpec((HID, HID), lambda i: (0, 0)),
            pl.BlockSpec((HID,), lambda i: (0,)),
            pl.BlockSpec((2 * HID, 1), lambda i: (0, 0)),
        ],
        out_specs=rb,
        out_shape=jax.ShapeDtypeStruct((N, HID), jnp.float32),
    )(X, h1, h2, h3, f1, f2, f3, f4, W1, b1, W2, b2, a)


def kernel(X, edge_index, W1, b1, W2, b2, a):
    row = edge_index[0]
    col = edge_index[1]
    pad = EPAD - EPT
    col_r = col.reshape(NTILES, EPT)
    spad = SNJ * CHUNK - ESUB
    col_s = jnp.pad(col.reshape(16, ESUB),
                    ((0, 0), (0, spad))).reshape(16, SNJ, CHUNK)
    row_s = jnp.pad(row.reshape(16, ESUB), ((0, 0), (0, spad)),
                    constant_values=N).reshape(16, SNJ, CHUNK)
    pk_g = (row_s << 14) | col_s
    col_d = jnp.pad(col_r, ((0, 0), (0, pad)),
                    constant_values=N).reshape(NTILES, NJ, CHUNK)
    zeros_slab = jnp.zeros((SLAB, HIDH), jnp.float32)
    ones_deg = jnp.ones((CHUNK, DEGW), jnp.float32)
    zeros_deg = jnp.zeros((SLAB, DEGW), jnp.float32)

    degp = _get_deg()(col_d, ones_deg, zeros_deg)
    deg = _degsum(degp)[:N]
    Dg, Dw, y, z = _prep(X, deg)

    spmm = _get_spmm()
    hs = []
    for _ in range(3):
        P = spmm(y, pk_g, zeros_slab)
        y, h = _gcn_update(P[:, :N], y, Dg)
        hs.append(h)

    fp = X
    fs = []
    for _ in range(4):
        P = spmm(z, pk_g, zeros_slab)
        fp, z = _sct_update(P[:, :N], fp, Dw)
        fs.append(fp)

    return _tail(X, hs[0], hs[1], hs[2], fs[0], fs[1], fs[2], fs[3],
                 W1, b1, W2, b2, a)
